# Initial kernel scaffold; baseline (speedup 1.0000x reference)
import jax
import jax.numpy as jnp
from jax.experimental import pallas as pl


def kernel(edge_attr, W0, W1, W2):
    raise NotImplementedError("write your pallas kernel here")


# speedup vs baseline: 1.0891x; 1.0891x over previous
import functools

import jax
import jax.numpy as jnp
from jax import lax
from jax.experimental import pallas as pl
from jax.experimental.pallas import tpu as pltpu
from jax.experimental.pallas import tpu_sc as plsc

D = 128
V0, V1, V2 = 5, 6, 2
NC, NS = 2, 16
NW = NC * NS


def _prep_body(c0_ref, c1_ref, c2_ref, w0_ref, w1_ref, w2_ref,
               codes_ref, t_ref):
    i0 = jnp.clip(c0_ref[...], 0, V0 - 1)
    i1 = jnp.clip(c1_ref[...], 0, V1 - 1)
    i2 = jnp.clip(c2_ref[...], 0, V2 - 1)
    codes_ref[...] = i0 * (V1 * V2) + i1 * V2 + i2

    w0 = w0_ref[...] * (1.0 / 3.0)
    w1 = w1_ref[...] * (1.0 / 3.0)
    w2 = w2_ref[...] * (1.0 / 3.0)
    for a in range(V0):
        for b in range(V1):
            row = (a * V1 + b) * V2
            t_ref[pl.ds(row, V2), :] = (w0[a:a + 1, :] + w1[b:b + 1, :]) + w2


def _make_sc_gather(n_groups):
    mesh = plsc.VectorSubcoreMesh(core_axis_name="c", subcore_axis_name="s")

    @functools.partial(
        pl.kernel,
        out_type=jax.ShapeDtypeStruct((n_groups * D, D), jnp.float32),
        mesh=mesh,
        scratch_types=[
            pltpu.VMEM((D,), jnp.int32),
            pltpu.VMEM((D, D), jnp.float32),
            pltpu.SemaphoreType.DMA,
        ],
    )
    def sc_gather(t_hbm, codes_hbm, out_hbm, idx_v, rows_v, sem):
        wid = lax.axis_index("s") * NC + lax.axis_index("c")

        def body(i, carry):
            g = wid + i * NW
            pltpu.sync_copy(codes_hbm.at[g], idx_v)
            pltpu.async_copy(t_hbm.at[idx_v], rows_v, sem).wait()
            pltpu.sync_copy(rows_v, out_hbm.at[pl.ds(g * D, D)])
            return carry

        n_i = (n_groups - wid + NW - 1) // NW
        lax.fori_loop(0, n_i, body, 0)

    return sc_gather


def kernel(edge_attr, W0, W1, W2):
    E = edge_attr.shape[0]
    n_groups = E // D
    assert n_groups * D == E

    ea = edge_attr.astype(jnp.int32)
    c0 = ea[:, 0].reshape(n_groups, D)
    c1 = ea[:, 1].reshape(n_groups, D)
    c2 = ea[:, 2].reshape(n_groups, D)

    codes2d, table = pl.pallas_call(
        _prep_body,
        out_shape=[
            jax.ShapeDtypeStruct((n_groups, D), jnp.int32),
            jax.ShapeDtypeStruct((V0 * V1 * V2, D), jnp.float32),
        ],
    )(c0, c1, c2, W0, W1, W2)

    return _make_sc_gather(n_groups)(table, codes2d)


# speedup vs baseline: 1.0992x; 1.0092x over previous
import functools

import jax
import jax.numpy as jnp
from jax import lax
from jax.experimental import pallas as pl
from jax.experimental.pallas import tpu as pltpu
from jax.experimental.pallas import tpu_sc as plsc

D = 128
V0, V1, V2 = 5, 6, 2
NC, NS = 2, 16
NW = NC * NS


def _prep_body(c0_ref, c1_ref, c2_ref, w0_ref, w1_ref, w2_ref,
               codes_ref, t_ref):
    i0 = jnp.clip(c0_ref[...], 0, V0 - 1)
    i1 = jnp.clip(c1_ref[...], 0, V1 - 1)
    i2 = jnp.clip(c2_ref[...], 0, V2 - 1)
    codes_ref[...] = i0 * (V1 * V2) + i1 * V2 + i2

    w0 = w0_ref[...] * (1.0 / 3.0)
    w1 = w1_ref[...] * (1.0 / 3.0)
    w2 = w2_ref[...] * (1.0 / 3.0)
    for a in range(V0):
        for b in range(V1):
            row = (a * V1 + b) * V2
            t_ref[pl.ds(row, V2), :] = (w0[a:a + 1, :] + w1[b:b + 1, :]) + w2


def _make_sc_gather(n_groups):
    mesh = plsc.VectorSubcoreMesh(core_axis_name="c", subcore_axis_name="s")
    per_w = n_groups // NW
    n_extra = n_groups - per_w * NW
    CH = 3
    ROWS = CH * D
    n_main = per_w // CH
    assert n_main * CH == per_w

    @functools.partial(
        pl.kernel,
        out_type=jax.ShapeDtypeStruct((n_groups * D, D), jnp.float32),
        mesh=mesh,
        scratch_types=[
            pltpu.VMEM(((per_w + 1) * D, ), jnp.int32),
            pltpu.VMEM((2, ROWS, D), jnp.float32),
            pltpu.SemaphoreType.DMA,
            pltpu.SemaphoreType.DMA,
            pltpu.SemaphoreType.DMA,
            pltpu.SemaphoreType.DMA,
        ],
    )
    def sc_gather(t_hbm, codes_hbm, out_hbm, idx_v, rows_v, g0, g1, o0, o1):
        wid = lax.axis_index("s") * NC + lax.axis_index("c")
        gbase = wid * per_w
        gsems = (g0, g1)
        osems = (o0, o1)

        pltpu.sync_copy(codes_hbm.at[pl.ds(gbase * D, per_w * D)],
                        idx_v.at[pl.ds(0, per_w * D)])

        def fire(k, s):
            for j in range(CH):
                pltpu.async_copy(t_hbm.at[idx_v.at[pl.ds((k * CH + j) * D, D)]],
                                 rows_v.at[s, pl.ds(j * D, D)], gsems[s])

        def drain_g(s):
            pltpu.make_async_copy(out_hbm.at[pl.ds(0, ROWS)],
                                  rows_v.at[s], gsems[s]).wait()

        def fire_out(k, s):
            pltpu.async_copy(rows_v.at[s],
                             out_hbm.at[pl.ds((gbase + k * CH) * D, ROWS)],
                             osems[s])

        def drain_out(s):
            pltpu.make_async_copy(rows_v.at[s],
                                  out_hbm.at[pl.ds(0, ROWS)], osems[s]).wait()

        fire(0, 0)
        fire(1, 1)

        def body(m, carry):
            k0 = 2 * m
            k1 = 2 * m + 1
            drain_g(0)
            fire_out(k0, 0)
            drain_g(1)
            fire_out(k1, 1)

            @pl.when(k0 + 2 < n_main)
            def _():
                drain_out(0)
                fire(k0 + 2, 0)

            @pl.when(k1 + 2 < n_main)
            def _():
                drain_out(1)
                fire(k1 + 2, 1)

            return carry

        lax.fori_loop(0, n_main // 2, body, 0)
        drain_out(0)
        drain_out(1)

        @pl.when(wid < n_extra)
        def _():
            g = NW * per_w + wid
            pltpu.sync_copy(codes_hbm.at[pl.ds(g * D, D)],
                            idx_v.at[pl.ds(per_w * D, D)])
            pltpu.async_copy(t_hbm.at[idx_v.at[pl.ds(per_w * D, D)]],
                             rows_v.at[0, pl.ds(0, D)], g0).wait()
            pltpu.sync_copy(rows_v.at[0, pl.ds(0, D)],
                            out_hbm.at[pl.ds(g * D, D)])

    return sc_gather


def kernel(edge_attr, W0, W1, W2):
    E = edge_attr.shape[0]
    n_groups = E // D
    assert n_groups * D == E

    ea = edge_attr.astype(jnp.int32)
    c0 = ea[:, 0].reshape(n_groups, D)
    c1 = ea[:, 1].reshape(n_groups, D)
    c2 = ea[:, 2].reshape(n_groups, D)

    codes2d, table = pl.pallas_call(
        _prep_body,
        out_shape=[
            jax.ShapeDtypeStruct((n_groups, D), jnp.int32),
            jax.ShapeDtypeStruct((V0 * V1 * V2, D), jnp.float32),
        ],
    )(c0, c1, c2, W0, W1, W2)

    return _make_sc_gather(n_groups)(table, codes2d.reshape(-1))


# speedup vs baseline: 5.1068x; 4.6460x over previous
import functools

import jax
import jax.numpy as jnp
from jax import lax
from jax.experimental import pallas as pl
from jax.experimental.pallas import tpu as pltpu
from jax.experimental.pallas import tpu_sc as plsc

D = 128
V0, V1, V2 = 5, 6, 2
NC, NS = 2, 16
NW = NC * NS


def _make_prep_body(per_w, n_low):
    def _prep_body(c0_ref, c1_ref, c2_ref, w0_ref, w1_ref, w2_ref,
                   codes_ref, t_ref):
        i0 = jnp.clip(c0_ref[...], 0, V0 - 1)
        i1 = jnp.clip(c1_ref[...], 0, V1 - 1)
        i2 = jnp.clip(c2_ref[...], 0, V2 - 1)
        code = i0 * (V1 * V2) + i1 * V2 + i2
        g = lax.broadcasted_iota(jnp.int32, code.shape, 0)
        owner = jnp.where(g < n_low, g // per_w, g - n_low)
        codes_ref[...] = code + owner * (V0 * V1 * V2)

        w0 = w0_ref[...] * (1.0 / 3.0)
        w1 = w1_ref[...] * (1.0 / 3.0)
        w2 = w2_ref[...] * (1.0 / 3.0)
        rows = []
        for a in range(V0):
            for b in range(V1):
                rows.append((w0[a:a + 1, :] + w1[b:b + 1, :]) + w2)
        t = jnp.concatenate(rows, axis=0)
        for w in range(NW):
            t_ref[pl.ds(w * (V0 * V1 * V2), V0 * V1 * V2), :] = t

    return _prep_body


def _make_sc_gather(n_groups):
    mesh = plsc.VectorSubcoreMesh(core_axis_name="c", subcore_axis_name="s")
    per_w = n_groups // NW
    n_extra = n_groups - per_w * NW
    CH = 3
    ROWS = CH * D
    n_main = per_w // CH
    assert n_main * CH == per_w

    @functools.partial(
        pl.kernel,
        out_type=jax.ShapeDtypeStruct((n_groups * D, D), jnp.float32),
        mesh=mesh,
        scratch_types=[
            pltpu.VMEM(((per_w + 1) * D, ), jnp.int32),
            pltpu.VMEM((2, ROWS, D), jnp.float32),
            pltpu.SemaphoreType.DMA,
            pltpu.SemaphoreType.DMA,
            pltpu.SemaphoreType.DMA,
            pltpu.SemaphoreType.DMA,
        ],
    )
    def sc_gather(t_hbm, codes_hbm, out_hbm, idx_v, rows_v, g0, g1, o0, o1):
        wid = lax.axis_index("s") * NC + lax.axis_index("c")
        gbase = wid * per_w
        gsems = (g0, g1)
        osems = (o0, o1)

        pltpu.sync_copy(codes_hbm.at[pl.ds(gbase * D, per_w * D)],
                        idx_v.at[pl.ds(0, per_w * D)])

        def fire(k, s):
            for j in range(CH):
                pltpu.async_copy(t_hbm.at[idx_v.at[pl.ds((k * CH + j) * D, D)]],
                                 rows_v.at[s, pl.ds(j * D, D)], gsems[s])

        def drain_g(s):
            pltpu.make_async_copy(out_hbm.at[pl.ds(0, ROWS)],
                                  rows_v.at[s], gsems[s]).wait()

        def fire_out(k, s):
            pltpu.async_copy(rows_v.at[s],
                             out_hbm.at[pl.ds((gbase + k * CH) * D, ROWS)],
                             osems[s])

        def drain_out(s):
            pltpu.make_async_copy(rows_v.at[s],
                                  out_hbm.at[pl.ds(0, ROWS)], osems[s]).wait()

        fire(0, 0)
        fire(1, 1)

        def body(m, carry):
            k0 = 2 * m
            k1 = 2 * m + 1
            drain_g(0)
            fire_out(k0, 0)
            drain_g(1)
            fire_out(k1, 1)

            @pl.when(k0 + 2 < n_main)
            def _():
                drain_out(0)
                fire(k0 + 2, 0)

            @pl.when(k1 + 2 < n_main)
            def _():
                drain_out(1)
                fire(k1 + 2, 1)

            return carry

        lax.fori_loop(0, n_main // 2, body, 0)
        drain_out(0)
        drain_out(1)

        @pl.when(wid < n_extra)
        def _():
            g = NW * per_w + wid
            pltpu.sync_copy(codes_hbm.at[pl.ds(g * D, D)],
                            idx_v.at[pl.ds(per_w * D, D)])
            pltpu.async_copy(t_hbm.at[idx_v.at[pl.ds(per_w * D, D)]],
                             rows_v.at[0, pl.ds(0, D)], g0).wait()
            pltpu.sync_copy(rows_v.at[0, pl.ds(0, D)],
                            out_hbm.at[pl.ds(g * D, D)])

    return sc_gather


def kernel(edge_attr, W0, W1, W2):
    E = edge_attr.shape[0]
    n_groups = E // D
    assert n_groups * D == E

    ea = edge_attr.astype(jnp.int32)
    c0 = ea[:, 0].reshape(n_groups, D)
    c1 = ea[:, 1].reshape(n_groups, D)
    c2 = ea[:, 2].reshape(n_groups, D)

    per_w = n_groups // NW
    codes2d, table = pl.pallas_call(
        _make_prep_body(per_w, per_w * NW),
        out_shape=[
            jax.ShapeDtypeStruct((n_groups, D), jnp.int32),
            jax.ShapeDtypeStruct((NW * V0 * V1 * V2, D), jnp.float32),
        ],
    )(c0, c1, c2, W0, W1, W2)

    return _make_sc_gather(n_groups)(table, codes2d.reshape(-1))


# speedup vs baseline: 8.3040x; 1.6261x over previous
import functools

import jax
import jax.numpy as jnp
from jax import lax
from jax.experimental import pallas as pl
from jax.experimental.pallas import tpu as pltpu
from jax.experimental.pallas import tpu_sc as plsc

D = 128
V0, V1, V2 = 5, 6, 2
NC, NS = 2, 16
NW = NC * NS
SUB = 4


def _make_prep_body(per_w, n_low):
    def _prep_body(c0_ref, c1_ref, c2_ref, w0_ref, w1_ref, w2_ref,
                   codes_ref, t_ref):
        i0 = jnp.clip(c0_ref[...], 0, V0 - 1)
        i1 = jnp.clip(c1_ref[...], 0, V1 - 1)
        i2 = jnp.clip(c2_ref[...], 0, V2 - 1)
        code = i0 * (V1 * V2) + i1 * V2 + i2
        g = lax.broadcasted_iota(jnp.int32, code.shape, 0)
        owner = jnp.where(g < n_low, g // per_w, g - n_low)
        lane = lax.broadcasted_iota(jnp.int32, code.shape, 1)
        rep = owner * SUB + (lane % SUB)
        codes_ref[...] = code + rep * (V0 * V1 * V2)

        w0 = w0_ref[...] * (1.0 / 3.0)
        w1 = w1_ref[...] * (1.0 / 3.0)
        w2 = w2_ref[...] * (1.0 / 3.0)
        rows = []
        for a in range(V0):
            for b in range(V1):
                rows.append((w0[a:a + 1, :] + w1[b:b + 1, :]) + w2)
        t = jnp.concatenate(rows, axis=0)
        for w in range(NW * SUB):
            t_ref[pl.ds(w * (V0 * V1 * V2), V0 * V1 * V2), :] = t

    return _prep_body


def _make_sc_gather(n_groups):
    mesh = plsc.VectorSubcoreMesh(core_axis_name="c", subcore_axis_name="s")
    per_w = n_groups // NW
    n_extra = n_groups - per_w * NW
    CH = 3
    ROWS = CH * D
    n_main = per_w // CH
    assert n_main * CH == per_w

    @functools.partial(
        pl.kernel,
        out_type=jax.ShapeDtypeStruct((n_groups * D, D), jnp.float32),
        mesh=mesh,
        scratch_types=[
            pltpu.VMEM(((per_w + 1) * D, ), jnp.int32),
            pltpu.VMEM((2, ROWS, D), jnp.float32),
            pltpu.SemaphoreType.DMA,
            pltpu.SemaphoreType.DMA,
            pltpu.SemaphoreType.DMA,
            pltpu.SemaphoreType.DMA,
        ],
    )
    def sc_gather(t_hbm, codes_hbm, out_hbm, idx_v, rows_v, g0, g1, o0, o1):
        wid = lax.axis_index("s") * NC + lax.axis_index("c")
        gbase = wid * per_w
        gsems = (g0, g1)
        osems = (o0, o1)

        pltpu.sync_copy(codes_hbm.at[pl.ds(gbase * D, per_w * D)],
                        idx_v.at[pl.ds(0, per_w * D)])

        def fire(k, s):
            for j in range(CH):
                pltpu.async_copy(t_hbm.at[idx_v.at[pl.ds((k * CH + j) * D, D)]],
                                 rows_v.at[s, pl.ds(j * D, D)], gsems[s])

        def drain_g(s):
            pltpu.make_async_copy(out_hbm.at[pl.ds(0, ROWS)],
                                  rows_v.at[s], gsems[s]).wait()

        def fire_out(k, s):
            pltpu.async_copy(rows_v.at[s],
                             out_hbm.at[pl.ds((gbase + k * CH) * D, ROWS)],
                             osems[s])

        def drain_out(s):
            pltpu.make_async_copy(rows_v.at[s],
                                  out_hbm.at[pl.ds(0, ROWS)], osems[s]).wait()

        fire(0, 0)
        fire(1, 1)

        def body(m, carry):
            k0 = 2 * m
            k1 = 2 * m + 1
            drain_g(0)
            fire_out(k0, 0)
            drain_g(1)
            fire_out(k1, 1)

            @pl.when(k0 + 2 < n_main)
            def _():
                drain_out(0)
                fire(k0 + 2, 0)

            @pl.when(k1 + 2 < n_main)
            def _():
                drain_out(1)
                fire(k1 + 2, 1)

            return carry

        lax.fori_loop(0, n_main // 2, body, 0)
        drain_out(0)
        drain_out(1)

        @pl.when(wid < n_extra)
        def _():
            g = NW * per_w + wid
            pltpu.sync_copy(codes_hbm.at[pl.ds(g * D, D)],
                            idx_v.at[pl.ds(per_w * D, D)])
            pltpu.async_copy(t_hbm.at[idx_v.at[pl.ds(per_w * D, D)]],
                             rows_v.at[0, pl.ds(0, D)], g0).wait()
            pltpu.sync_copy(rows_v.at[0, pl.ds(0, D)],
                            out_hbm.at[pl.ds(g * D, D)])

    return sc_gather


def kernel(edge_attr, W0, W1, W2):
    E = edge_attr.shape[0]
    n_groups = E // D
    assert n_groups * D == E

    ea = edge_attr.astype(jnp.int32)
    c0 = ea[:, 0].reshape(n_groups, D)
    c1 = ea[:, 1].reshape(n_groups, D)
    c2 = ea[:, 2].reshape(n_groups, D)

    per_w = n_groups // NW
    codes2d, table = pl.pallas_call(
        _make_prep_body(per_w, per_w * NW),
        out_shape=[
            jax.ShapeDtypeStruct((n_groups, D), jnp.int32),
            jax.ShapeDtypeStruct((NW * SUB * V0 * V1 * V2, D), jnp.float32),
        ],
    )(c0, c1, c2, W0, W1, W2)

    return _make_sc_gather(n_groups)(table, codes2d.reshape(-1))


# speedup vs baseline: 8.6365x; 1.0400x over previous
import functools

import jax
import jax.numpy as jnp
from jax import lax
from jax.experimental import pallas as pl
from jax.experimental.pallas import tpu as pltpu
from jax.experimental.pallas import tpu_sc as plsc

D = 128
V0, V1, V2 = 5, 6, 2
NC, NS = 2, 16
NW = NC * NS
SUB = 8


def _make_prep_body(per_w, n_low):
    def _prep_body(c0_ref, c1_ref, c2_ref, w0_ref, w1_ref, w2_ref,
                   codes_ref, t_ref):
        i0 = jnp.clip(c0_ref[...], 0, V0 - 1)
        i1 = jnp.clip(c1_ref[...], 0, V1 - 1)
        i2 = jnp.clip(c2_ref[...], 0, V2 - 1)
        code = i0 * (V1 * V2) + i1 * V2 + i2
        g = lax.broadcasted_iota(jnp.int32, code.shape, 0)
        owner = jnp.where(g < n_low, g // per_w, g - n_low)
        lane = lax.broadcasted_iota(jnp.int32, code.shape, 1)
        rep = owner * SUB + (lane % SUB)
        codes_ref[...] = code + rep * (V0 * V1 * V2)

        w0 = w0_ref[...] * (1.0 / 3.0)
        w1 = w1_ref[...] * (1.0 / 3.0)
        w2 = w2_ref[...] * (1.0 / 3.0)
        rows = []
        for a in range(V0):
            for b in range(V1):
                rows.append((w0[a:a + 1, :] + w1[b:b + 1, :]) + w2)
        t = jnp.concatenate(rows, axis=0)
        for w in range(NW * SUB):
            t_ref[pl.ds(w * (V0 * V1 * V2), V0 * V1 * V2), :] = t

    return _prep_body


def _make_sc_gather(n_groups):
    mesh = plsc.VectorSubcoreMesh(core_axis_name="c", subcore_axis_name="s")
    per_w = n_groups // NW
    n_extra = n_groups - per_w * NW
    CH = 3
    ROWS = CH * D
    n_main = per_w // CH
    assert n_main * CH == per_w

    @functools.partial(
        pl.kernel,
        out_type=jax.ShapeDtypeStruct((n_groups * D, D), jnp.float32),
        mesh=mesh,
        scratch_types=[
            pltpu.VMEM(((per_w + 1) * D, ), jnp.int32),
            pltpu.VMEM((2, ROWS, D), jnp.float32),
            pltpu.SemaphoreType.DMA,
            pltpu.SemaphoreType.DMA,
            pltpu.SemaphoreType.DMA,
            pltpu.SemaphoreType.DMA,
        ],
    )
    def sc_gather(t_hbm, codes_hbm, out_hbm, idx_v, rows_v, g0, g1, o0, o1):
        wid = lax.axis_index("s") * NC + lax.axis_index("c")
        gbase = wid * per_w
        gsems = (g0, g1)
        osems = (o0, o1)

        pltpu.sync_copy(codes_hbm.at[pl.ds(gbase * D, per_w * D)],
                        idx_v.at[pl.ds(0, per_w * D)])

        def fire(k, s):
            for j in range(CH):
                pltpu.async_copy(t_hbm.at[idx_v.at[pl.ds((k * CH + j) * D, D)]],
                                 rows_v.at[s, pl.ds(j * D, D)], gsems[s])

        def drain_g(s):
            pltpu.make_async_copy(out_hbm.at[pl.ds(0, ROWS)],
                                  rows_v.at[s], gsems[s]).wait()

        def fire_out(k, s):
            pltpu.async_copy(rows_v.at[s],
                             out_hbm.at[pl.ds((gbase + k * CH) * D, ROWS)],
                             osems[s])

        def drain_out(s):
            pltpu.make_async_copy(rows_v.at[s],
                                  out_hbm.at[pl.ds(0, ROWS)], osems[s]).wait()

        fire(0, 0)
        fire(1, 1)

        def body(m, carry):
            k0 = 2 * m
            k1 = 2 * m + 1
            drain_g(0)
            fire_out(k0, 0)
            drain_g(1)
            fire_out(k1, 1)

            @pl.when(k0 + 2 < n_main)
            def _():
                drain_out(0)
                fire(k0 + 2, 0)

            @pl.when(k1 + 2 < n_main)
            def _():
                drain_out(1)
                fire(k1 + 2, 1)

            return carry

        lax.fori_loop(0, n_main // 2, body, 0)
        drain_out(0)
        drain_out(1)

        @pl.when(wid < n_extra)
        def _():
            g = NW * per_w + wid
            pltpu.sync_copy(codes_hbm.at[pl.ds(g * D, D)],
                            idx_v.at[pl.ds(per_w * D, D)])
            pltpu.async_copy(t_hbm.at[idx_v.at[pl.ds(per_w * D, D)]],
                             rows_v.at[0, pl.ds(0, D)], g0).wait()
            pltpu.sync_copy(rows_v.at[0, pl.ds(0, D)],
                            out_hbm.at[pl.ds(g * D, D)])

    return sc_gather


def kernel(edge_attr, W0, W1, W2):
    E = edge_attr.shape[0]
    n_groups = E // D
    assert n_groups * D == E

    ea = edge_attr.astype(jnp.int32)
    c0 = ea[:, 0].reshape(n_groups, D)
    c1 = ea[:, 1].reshape(n_groups, D)
    c2 = ea[:, 2].reshape(n_groups, D)

    per_w = n_groups // NW
    codes2d, table = pl.pallas_call(
        _make_prep_body(per_w, per_w * NW),
        out_shape=[
            jax.ShapeDtypeStruct((n_groups, D), jnp.int32),
            jax.ShapeDtypeStruct((NW * SUB * V0 * V1 * V2, D), jnp.float32),
        ],
    )(c0, c1, c2, W0, W1, W2)

    return _make_sc_gather(n_groups)(table, codes2d.reshape(-1))


# speedup vs baseline: 10.4665x; 1.2119x over previous
import functools

import jax
import jax.numpy as jnp
from jax import lax
from jax.experimental import pallas as pl
from jax.experimental.pallas import tpu as pltpu
from jax.experimental.pallas import tpu_sc as plsc

D = 128
V0, V1, V2 = 5, 6, 2
NC, NS = 2, 16
NW = NC * NS
SUB = 16


def _make_prep_body(per_w, n_low):
    def _prep_body(c0_ref, c1_ref, c2_ref, w0_ref, w1_ref, w2_ref,
                   codes_ref, t_ref):
        i0 = jnp.clip(c0_ref[...], 0, V0 - 1)
        i1 = jnp.clip(c1_ref[...], 0, V1 - 1)
        i2 = jnp.clip(c2_ref[...], 0, V2 - 1)
        code = i0 * (V1 * V2) + i1 * V2 + i2
        g = lax.broadcasted_iota(jnp.int32, code.shape, 0)
        owner = jnp.where(g < n_low, g // per_w, g - n_low)
        lane = lax.broadcasted_iota(jnp.int32, code.shape, 1)
        rep = owner * SUB + (lane % SUB)
        codes_ref[...] = code + rep * (V0 * V1 * V2)

        w0 = w0_ref[...] * (1.0 / 3.0)
        w1 = w1_ref[...] * (1.0 / 3.0)
        w2 = w2_ref[...] * (1.0 / 3.0)
        rows = []
        for a in range(V0):
            for b in range(V1):
                rows.append((w0[a:a + 1, :] + w1[b:b + 1, :]) + w2)
        t = jnp.concatenate(rows, axis=0)
        for w in range(NW * SUB):
            t_ref[pl.ds(w * (V0 * V1 * V2), V0 * V1 * V2), :] = t

    return _prep_body


def _make_sc_gather(n_groups):
    mesh = plsc.VectorSubcoreMesh(core_axis_name="c", subcore_axis_name="s")
    per_w = n_groups // NW
    n_extra = n_groups - per_w * NW
    CH = 3
    ROWS = CH * D
    n_main = per_w // CH
    assert n_main * CH == per_w

    @functools.partial(
        pl.kernel,
        out_type=jax.ShapeDtypeStruct((n_groups * D, D), jnp.float32),
        mesh=mesh,
        scratch_types=[
            pltpu.VMEM(((per_w + 1) * D, ), jnp.int32),
            pltpu.VMEM((2, ROWS, D), jnp.float32),
            pltpu.SemaphoreType.DMA,
            pltpu.SemaphoreType.DMA,
            pltpu.SemaphoreType.DMA,
            pltpu.SemaphoreType.DMA,
        ],
    )
    def sc_gather(t_hbm, codes_hbm, out_hbm, idx_v, rows_v, g0, g1, o0, o1):
        wid = lax.axis_index("s") * NC + lax.axis_index("c")
        gbase = wid * per_w
        gsems = (g0, g1)
        osems = (o0, o1)

        pltpu.sync_copy(codes_hbm.at[pl.ds(gbase * D, per_w * D)],
                        idx_v.at[pl.ds(0, per_w * D)])

        def fire(k, s):
            for j in range(CH):
                pltpu.async_copy(t_hbm.at[idx_v.at[pl.ds((k * CH + j) * D, D)]],
                                 rows_v.at[s, pl.ds(j * D, D)], gsems[s])

        def drain_g(s):
            pltpu.make_async_copy(out_hbm.at[pl.ds(0, ROWS)],
                                  rows_v.at[s], gsems[s]).wait()

        def fire_out(k, s):
            pltpu.async_copy(rows_v.at[s],
                             out_hbm.at[pl.ds((gbase + k * CH) * D, ROWS)],
                             osems[s])

        def drain_out(s):
            pltpu.make_async_copy(rows_v.at[s],
                                  out_hbm.at[pl.ds(0, ROWS)], osems[s]).wait()

        fire(0, 0)
        fire(1, 1)

        def body(m, carry):
            k0 = 2 * m
            k1 = 2 * m + 1
            drain_g(0)
            fire_out(k0, 0)
            drain_g(1)
            fire_out(k1, 1)

            @pl.when(k0 + 2 < n_main)
            def _():
                drain_out(0)
                fire(k0 + 2, 0)

            @pl.when(k1 + 2 < n_main)
            def _():
                drain_out(1)
                fire(k1 + 2, 1)

            return carry

        lax.fori_loop(0, n_main // 2, body, 0)
        drain_out(0)
        drain_out(1)

        @pl.when(wid < n_extra)
        def _():
            g = NW * per_w + wid
            pltpu.sync_copy(codes_hbm.at[pl.ds(g * D, D)],
                            idx_v.at[pl.ds(per_w * D, D)])
            pltpu.async_copy(t_hbm.at[idx_v.at[pl.ds(per_w * D, D)]],
                             rows_v.at[0, pl.ds(0, D)], g0).wait()
            pltpu.sync_copy(rows_v.at[0, pl.ds(0, D)],
                            out_hbm.at[pl.ds(g * D, D)])

    return sc_gather


def kernel(edge_attr, W0, W1, W2):
    E = edge_attr.shape[0]
    n_groups = E // D
    assert n_groups * D == E

    ea = edge_attr.astype(jnp.int32)
    c0 = ea[:, 0].reshape(n_groups, D)
    c1 = ea[:, 1].reshape(n_groups, D)
    c2 = ea[:, 2].reshape(n_groups, D)

    per_w = n_groups // NW
    codes2d, table = pl.pallas_call(
        _make_prep_body(per_w, per_w * NW),
        out_shape=[
            jax.ShapeDtypeStruct((n_groups, D), jnp.int32),
            jax.ShapeDtypeStruct((NW * SUB * V0 * V1 * V2, D), jnp.float32),
        ],
    )(c0, c1, c2, W0, W1, W2)

    return _make_sc_gather(n_groups)(table, codes2d.reshape(-1))


# speedup vs baseline: 10.7810x; 1.0300x over previous
import functools

import jax
import jax.numpy as jnp
from jax import lax
from jax.experimental import pallas as pl
from jax.experimental.pallas import tpu as pltpu
from jax.experimental.pallas import tpu_sc as plsc

D = 128
V0, V1, V2 = 5, 6, 2
NC, NS = 2, 16
NW = NC * NS
SUB = 32


def _make_prep_body(per_w, n_low):
    def _prep_body(c0_ref, c1_ref, c2_ref, w0_ref, w1_ref, w2_ref,
                   codes_ref, t_ref):
        i0 = jnp.clip(c0_ref[...], 0, V0 - 1)
        i1 = jnp.clip(c1_ref[...], 0, V1 - 1)
        i2 = jnp.clip(c2_ref[...], 0, V2 - 1)
        code = i0 * (V1 * V2) + i1 * V2 + i2
        g = lax.broadcasted_iota(jnp.int32, code.shape, 0)
        owner = jnp.where(g < n_low, g // per_w, g - n_low)
        lane = lax.broadcasted_iota(jnp.int32, code.shape, 1)
        rep = owner * SUB + (lane % SUB)
        codes_ref[...] = code + rep * (V0 * V1 * V2)

        w0 = w0_ref[...] * (1.0 / 3.0)
        w1 = w1_ref[...] * (1.0 / 3.0)
        w2 = w2_ref[...] * (1.0 / 3.0)
        rows = []
        for a in range(V0):
            for b in range(V1):
                rows.append((w0[a:a + 1, :] + w1[b:b + 1, :]) + w2)
        t = jnp.concatenate(rows, axis=0)
        for w in range(NW * SUB):
            t_ref[pl.ds(w * (V0 * V1 * V2), V0 * V1 * V2), :] = t

    return _prep_body


def _make_sc_gather(n_groups):
    mesh = plsc.VectorSubcoreMesh(core_axis_name="c", subcore_axis_name="s")
    per_w = n_groups // NW
    n_extra = n_groups - per_w * NW
    CH = 3
    ROWS = CH * D
    n_main = per_w // CH
    assert n_main * CH == per_w

    @functools.partial(
        pl.kernel,
        out_type=jax.ShapeDtypeStruct((n_groups * D, D), jnp.float32),
        mesh=mesh,
        scratch_types=[
            pltpu.VMEM(((per_w + 1) * D, ), jnp.int32),
            pltpu.VMEM((2, ROWS, D), jnp.float32),
            pltpu.SemaphoreType.DMA,
            pltpu.SemaphoreType.DMA,
            pltpu.SemaphoreType.DMA,
            pltpu.SemaphoreType.DMA,
        ],
    )
    def sc_gather(t_hbm, codes_hbm, out_hbm, idx_v, rows_v, g0, g1, o0, o1):
        wid = lax.axis_index("s") * NC + lax.axis_index("c")
        gbase = wid * per_w
        gsems = (g0, g1)
        osems = (o0, o1)

        pltpu.sync_copy(codes_hbm.at[pl.ds(gbase * D, per_w * D)],
                        idx_v.at[pl.ds(0, per_w * D)])

        def fire(k, s):
            for j in range(CH):
                pltpu.async_copy(t_hbm.at[idx_v.at[pl.ds((k * CH + j) * D, D)]],
                                 rows_v.at[s, pl.ds(j * D, D)], gsems[s])

        def drain_g(s):
            pltpu.make_async_copy(out_hbm.at[pl.ds(0, ROWS)],
                                  rows_v.at[s], gsems[s]).wait()

        def fire_out(k, s):
            pltpu.async_copy(rows_v.at[s],
                             out_hbm.at[pl.ds((gbase + k * CH) * D, ROWS)],
                             osems[s])

        def drain_out(s):
            pltpu.make_async_copy(rows_v.at[s],
                                  out_hbm.at[pl.ds(0, ROWS)], osems[s]).wait()

        fire(0, 0)
        fire(1, 1)

        def body(m, carry):
            k0 = 2 * m
            k1 = 2 * m + 1
            drain_g(0)
            fire_out(k0, 0)
            drain_g(1)
            fire_out(k1, 1)

            @pl.when(k0 + 2 < n_main)
            def _():
                drain_out(0)
                fire(k0 + 2, 0)

            @pl.when(k1 + 2 < n_main)
            def _():
                drain_out(1)
                fire(k1 + 2, 1)

            return carry

        lax.fori_loop(0, n_main // 2, body, 0)
        drain_out(0)
        drain_out(1)

        @pl.when(wid < n_extra)
        def _():
            g = NW * per_w + wid
            pltpu.sync_copy(codes_hbm.at[pl.ds(g * D, D)],
                            idx_v.at[pl.ds(per_w * D, D)])
            pltpu.async_copy(t_hbm.at[idx_v.at[pl.ds(per_w * D, D)]],
                             rows_v.at[0, pl.ds(0, D)], g0).wait()
            pltpu.sync_copy(rows_v.at[0, pl.ds(0, D)],
                            out_hbm.at[pl.ds(g * D, D)])

    return sc_gather


def kernel(edge_attr, W0, W1, W2):
    E = edge_attr.shape[0]
    n_groups = E // D
    assert n_groups * D == E

    ea = edge_attr.astype(jnp.int32)
    c0 = ea[:, 0].reshape(n_groups, D)
    c1 = ea[:, 1].reshape(n_groups, D)
    c2 = ea[:, 2].reshape(n_groups, D)

    per_w = n_groups // NW
    codes2d, table = pl.pallas_call(
        _make_prep_body(per_w, per_w * NW),
        out_shape=[
            jax.ShapeDtypeStruct((n_groups, D), jnp.int32),
            jax.ShapeDtypeStruct((NW * SUB * V0 * V1 * V2, D), jnp.float32),
        ],
    )(c0, c1, c2, W0, W1, W2)

    return _make_sc_gather(n_groups)(table, codes2d.reshape(-1))


# speedup vs baseline: 11.3735x; 1.0550x over previous
import functools

import jax
import jax.numpy as jnp
from jax import lax
from jax.experimental import pallas as pl
from jax.experimental.pallas import tpu as pltpu
from jax.experimental.pallas import tpu_sc as plsc

D = 128
V0, V1, V2 = 5, 6, 2
NC, NS = 2, 16
NW = NC * NS
SUB = 32


def _make_prep_body(per_w, n_low):
    def _prep_body(c0_ref, c1_ref, c2_ref, w0_ref, w1_ref, w2_ref,
                   codes_ref, t_ref):
        i0 = jnp.clip(c0_ref[...], 0, V0 - 1)
        i1 = jnp.clip(c1_ref[...], 0, V1 - 1)
        i2 = jnp.clip(c2_ref[...], 0, V2 - 1)
        code = i0 * (V1 * V2) + i1 * V2 + i2
        g = lax.broadcasted_iota(jnp.int32, code.shape, 0)
        owner = jnp.where(g < n_low, g // per_w, g - n_low)
        lane = lax.broadcasted_iota(jnp.int32, code.shape, 1)
        rep = owner * SUB + (lane % SUB)
        codes_ref[...] = code + rep * (V0 * V1 * V2)

        w0 = w0_ref[...] * (1.0 / 3.0)
        w1 = w1_ref[...] * (1.0 / 3.0)
        w2 = w2_ref[...] * (1.0 / 3.0)
        rows = []
        for a in range(V0):
            for b in range(V1):
                rows.append((w0[a:a + 1, :] + w1[b:b + 1, :]) + w2)
        t = jnp.concatenate(rows, axis=0)
        for w in range(NW * SUB):
            t_ref[pl.ds(w * (V0 * V1 * V2), V0 * V1 * V2), :] = t

    return _prep_body


def _make_sc_gather(n_groups):
    mesh = plsc.VectorSubcoreMesh(core_axis_name="c", subcore_axis_name="s")
    per_w = n_groups // NW
    n_extra = n_groups - per_w * NW
    CH = 2
    NSLOT = 3
    ROWS = CH * D
    n_main = per_w // CH
    assert n_main * CH == per_w

    @functools.partial(
        pl.kernel,
        out_type=jax.ShapeDtypeStruct((n_groups * D, D), jnp.float32),
        mesh=mesh,
        scratch_types=[
            pltpu.VMEM(((per_w + 1) * D, ), jnp.int32),
            pltpu.VMEM((NSLOT, ROWS, D), jnp.float32),
            pltpu.SemaphoreType.DMA,
            pltpu.SemaphoreType.DMA,
            pltpu.SemaphoreType.DMA,
            pltpu.SemaphoreType.DMA,
            pltpu.SemaphoreType.DMA,
            pltpu.SemaphoreType.DMA,
        ],
    )
    def sc_gather(t_hbm, codes_hbm, out_hbm, idx_v, rows_v,
                  g0, g1, g2, o0, o1, o2):
        wid = lax.axis_index("s") * NC + lax.axis_index("c")
        gbase = wid * per_w
        gsems = (g0, g1, g2)
        osems = (o0, o1, o2)

        pltpu.sync_copy(codes_hbm.at[pl.ds(gbase * D, per_w * D)],
                        idx_v.at[pl.ds(0, per_w * D)])

        def fire(k, s):
            for j in range(CH):
                pltpu.async_copy(t_hbm.at[idx_v.at[pl.ds((k * CH + j) * D, D)]],
                                 rows_v.at[s, pl.ds(j * D, D)], gsems[s])

        def drain_g(s):
            pltpu.make_async_copy(out_hbm.at[pl.ds(0, ROWS)],
                                  rows_v.at[s], gsems[s]).wait()

        def fire_out(k, s):
            pltpu.async_copy(rows_v.at[s],
                             out_hbm.at[pl.ds((gbase + k * CH) * D, ROWS)],
                             osems[s])

        def drain_out(s):
            pltpu.make_async_copy(rows_v.at[s],
                                  out_hbm.at[pl.ds(0, ROWS)], osems[s]).wait()

        fire(0, 0)
        fire(1, 1)

        def body(m, carry):
            for t in range(NSLOT):
                k = m * NSLOT + t
                drain_g(t)
                fire_out(k, t)
                s2 = (t + 2) % NSLOT

                @pl.when(k + 2 < n_main)
                def _():
                    @pl.when(k >= 1)
                    def _():
                        drain_out(s2)
                    fire(k + 2, s2)

            return carry

        assert n_main % NSLOT == 0
        lax.fori_loop(0, n_main // NSLOT, body, 0)
        for s in range(NSLOT):
            drain_out(s)

        @pl.when(wid < n_extra)
        def _():
            g = NW * per_w + wid
            pltpu.sync_copy(codes_hbm.at[pl.ds(g * D, D)],
                            idx_v.at[pl.ds(per_w * D, D)])
            pltpu.async_copy(t_hbm.at[idx_v.at[pl.ds(per_w * D, D)]],
                             rows_v.at[0, pl.ds(0, D)], g0).wait()
            pltpu.sync_copy(rows_v.at[0, pl.ds(0, D)],
                            out_hbm.at[pl.ds(g * D, D)])

    return sc_gather


def kernel(edge_attr, W0, W1, W2):
    E = edge_attr.shape[0]
    n_groups = E // D
    assert n_groups * D == E

    ea = edge_attr.astype(jnp.int32)
    c0 = ea[:, 0].reshape(n_groups, D)
    c1 = ea[:, 1].reshape(n_groups, D)
    c2 = ea[:, 2].reshape(n_groups, D)

    per_w = n_groups // NW
    codes2d, table = pl.pallas_call(
        _make_prep_body(per_w, per_w * NW),
        out_shape=[
            jax.ShapeDtypeStruct((n_groups, D), jnp.int32),
            jax.ShapeDtypeStruct((NW * SUB * V0 * V1 * V2, D), jnp.float32),
        ],
    )(c0, c1, c2, W0, W1, W2)

    return _make_sc_gather(n_groups)(table, codes2d.reshape(-1))
